# Initial kernel scaffold; baseline (speedup 1.0000x reference)
import jax
import jax.numpy as jnp
from jax.experimental import pallas as pl


def kernel(x, feature_embedding, linear_table, bias):
    raise NotImplementedError("write your pallas kernel here")


# speedup vs baseline: 2.3065x; 2.3065x over previous
import functools

import jax
import jax.numpy as jnp
from jax import lax
from jax.experimental import pallas as pl
from jax.experimental.pallas import tpu as pltpu
from jax.experimental.pallas import tpu_sc as plsc

B = 16384
F = 26
D = 32
F_TAB = 1000000
NC, NS, L = 2, 16, 16
NW = NC * NS
BPW = B // NW
CH = 64
NCH = BPW // CH
ROWS = CH * F
NJ = ROWS // 128

QS = 262144
TCB = 4096
TGRID = QS // TCB
NTB = -(-F_TAB // TCB)
LINB = 4 * QS // TGRID
NLB = -(-F_TAB // LINB)


def _transpose_body(x0, x1, x2, x3, xl, o_ref, o_lin):
    for q, xr in enumerate((x0, x1, x2, x3)):
        o_ref[:, q * D:(q + 1) * D] = jnp.transpose(xr[...], (1, 0))
    o_lin[...] = xl[0]


def _fm_body(x1d, xl1d, emb, lin, bias_h, out, idx_v, idx2_v, rows_v, linr_v,
             out_v, bias_v, sem, sem2):
    cid = lax.axis_index("c")
    sid = lax.axis_index("s")
    wid = sid * NC + cid
    pltpu.sync_copy(bias_h, bias_v.at[pl.ds(0, 1)])
    b0 = bias_v[...][0]
    li = jax.lax.iota(jnp.int32, 16)

    def fire(buf, c):
        flat0 = wid * (BPW * F) + c * ROWS
        pltpu.sync_copy(x1d.at[pl.ds(flat0, ROWS)], idx_v.at[buf])
        pltpu.sync_copy(xl1d.at[pl.ds(flat0, ROWS)], idx2_v.at[buf])
        for j in range(NJ):
            ij = idx_v.at[buf, pl.ds(j * 128, 128)]
            il = idx2_v.at[buf, pl.ds(j * 128, 128)]
            pltpu.async_copy(emb.at[ij], rows_v.at[buf, pl.ds(j * 128, 128)],
                             sem.at[buf])
            pltpu.async_copy(lin.at[il], linr_v.at[buf, pl.ds(j * 128, 128)],
                             sem2.at[buf])

    def drain(buf):
        for j in range(NJ):
            pltpu.make_async_copy(
                emb.at[idx_v.at[buf, pl.ds(j * 128, 128)]],
                rows_v.at[buf, pl.ds(j * 128, 128)], sem.at[buf]).wait()
            pltpu.make_async_copy(
                lin.at[idx2_v.at[buf, pl.ds(j * 128, 128)]],
                linr_v.at[buf, pl.ds(j * 128, 128)], sem2.at[buf]).wait()

    def compute(buf, c):
        rv = rows_v.at[buf]
        lv = linr_v.at[buf]
        for g in range(CH // L):
            r0 = li * F + (g * L * F)
            lin_acc = jnp.zeros((16,), jnp.float32)
            for f in range(F):
                lin_acc = lin_acc + plsc.load_gather(lv, [r0 + f])

            def dbody(d, ix):
                dv = jnp.full((16,), d, jnp.int32)
                acc = jnp.zeros((16,), jnp.float32)
                sq = jnp.zeros((16,), jnp.float32)
                for f in range(F):
                    v = plsc.load_gather(rv, [r0 + f, dv])
                    acc = acc + v
                    sq = sq + v * v
                return ix + acc * acc - sq

            ix = lax.fori_loop(0, D, dbody, jnp.zeros((16,), jnp.float32))
            logit = b0 + lin_acc + 0.5 * ix
            out_v[pl.ds(g * L, L)] = 1.0 / (1.0 + jnp.exp(-logit))

        pltpu.sync_copy(out_v, out.at[pl.ds(wid * BPW + c * CH, CH)])

    fire(0, 0)

    @pl.loop(0, NCH - 2, step=2)
    def chunk_loop(c):
        fire(1, c + 1)
        drain(0)
        compute(0, c)
        fire(0, c + 2)
        drain(1)
        compute(1, c + 1)

    fire(1, NCH - 1)
    drain(0)
    compute(0, NCH - 2)
    drain(1)
    compute(1, NCH - 1)


@functools.partial(jax.jit, static_argnames=())
def kernel(x, feature_embedding, linear_table, bias):
    xi = x.astype(jnp.int32)
    xk = ((xi & (QS - 1)) << 2) | (xi >> 18)
    x1d = jnp.reshape(xk, (B * F,))
    xlin1d = jnp.reshape(xi, (B * F,))
    embT = jnp.transpose(feature_embedding)
    linT = jnp.transpose(linear_table)
    emb_rm, lin_flat = pl.pallas_call(
        _transpose_body,
        grid=(TGRID,),
        in_specs=[
            pl.BlockSpec(
                (D, TCB),
                lambda i, q=q: (0, jnp.minimum(q * TGRID + i, NTB - 1)))
            for q in range(4)
        ] + [
            pl.BlockSpec((1, LINB), lambda i: (0, jnp.minimum(i, NLB - 1)))
        ],
        out_specs=[
            pl.BlockSpec((TCB, 128), lambda i: (i, 0)),
            pl.BlockSpec((LINB,), lambda i: (i,)),
        ],
        out_shape=[
            jax.ShapeDtypeStruct((QS, 128), jnp.float32),
            jax.ShapeDtypeStruct((4 * QS,), jnp.float32),
        ],
    )(embT, embT, embT, embT, linT)
    emb_lin = jnp.reshape(emb_rm, (4 * QS, D))
    lin1d = lin_flat

    mesh = plsc.VectorSubcoreMesh(core_axis_name="c", subcore_axis_name="s")
    fm = pl.kernel(
        _fm_body,
        out_type=jax.ShapeDtypeStruct((B,), jnp.float32),
        mesh=mesh,
        scratch_types=[
            pltpu.VMEM((2, ROWS), jnp.int32),
            pltpu.VMEM((2, ROWS), jnp.int32),
            pltpu.VMEM((2, ROWS, D), jnp.float32),
            pltpu.VMEM((2, ROWS), jnp.float32),
            pltpu.VMEM((CH,), jnp.float32),
            pltpu.VMEM((16,), jnp.float32),
            pltpu.SemaphoreType.DMA((2,)),
            pltpu.SemaphoreType.DMA((2,)),
        ],
        compiler_params=pltpu.CompilerParams(needs_layout_passes=False,
                                             use_tc_tiling_on_sc=False),
    )
    out = fm(x1d, xlin1d, emb_lin, lin1d, bias)
    return jnp.reshape(out, (B, 1))
